# baseline (speedup 1.0000x reference)
import jax
import jax.numpy as jnp
from jax import lax
from jax.experimental import pallas as pl
from jax.experimental.pallas import tpu as pltpu
from jax.experimental.pallas import tpu_sc as plsc

CAP = 100000
DIM = 64
BATCH = 16384

CAPP = CAP // 2
BATP = BATCH // 2
PDIM = 2 * DIM

NW = 32
SLABP = 1568
SLABP_LAST = CAPP - (NW - 1) * SLABP
LANES = 16
POS_PAD = 2 * SLABP
NG_EXT = SLABP // LANES

PCH = 64
MAXCH = (SLABP + PCH - 1) // PCH
LIST_PAD = MAXCH * PCH
NG_LIST = LIST_PAD // LANES

CCHP = 112
NCC = SLABP // CCHP
NCC_LAST = NCC - 2
CTAILP_LAST = SLABP_LAST - NCC_LAST * CCHP

_INT_MIN = -2147483647 - 1


def _sc_body(mem_hbm, idx_hbm, val_hbm, out_hbm,
             idx_v, pos_v, q1_v, wl1_v, wr1_v, qp2_v, wlp2_v, wrp2_v,
             gl_v, gr_v, mb_v, cbuf_v,
             isem0, isem1, osem0, osem1, gsem0, gsem1, ssem0, ssem1):
    wid = lax.axis_index("s") * 2 + lax.axis_index("c")
    basep = wid * SLABP
    base = 2 * basep
    is_last = wid == NW - 1
    slab_len = jnp.where(is_last, 2 * SLABP_LAST, 2 * SLABP)

    isems = (isem0, isem1)
    osems = (osem0, osem1)
    gsems = (gsem0, gsem1)
    ssems = (ssem0, ssem1)

    def cin(c, n):
        b = c % 2
        return pltpu.make_async_copy(
            mem_hbm.at[pl.ds(basep + c * CCHP, n)],
            cbuf_v.at[b, pl.ds(0, n)], isems[b])

    def cout(c, n):
        b = c % 2
        return pltpu.make_async_copy(
            cbuf_v.at[b, pl.ds(0, n)],
            out_hbm.at[pl.ds(basep + c * CCHP, n)], osems[b])

    cin(0, CCHP).start()
    cin(1, CCHP).start()

    with jax.named_scope("stage_idx"):
        pltpu.sync_copy(idx_hbm, idx_v)

    iota = lax.iota(jnp.int32, LANES)
    neg1 = jnp.full((LANES,), -1, jnp.int32)

    with jax.named_scope("init_pos"):
        @pl.loop(0, POS_PAD, step=LANES)
        def _(off):
            pos_v[pl.ds(off, LANES)] = neg1

    with jax.named_scope("scan"):
        @pl.loop(0, BATCH, step=LANES)
        def _(off):
            v = idx_v[pl.ds(off, LANES)]
            loc = v - base
            m = (loc >= 0) & (loc < slab_len)
            loc = jnp.where(m, loc, 0)
            slot = (loc >> 1) + (loc & 1) * SLABP
            plsc.store_scatter(pos_v, [slot], iota + off, mask=m)

    def _extract(g, cnt):
        pL = pos_v[pl.ds(g * LANES, LANES)]
        pR = pos_v[pl.ds(SLABP + g * LANES, LANES)]
        m = (pL >= 0) | (pR >= 0)
        qv = iota + (basep + g * LANES)
        plsc.store_compressed(q1_v.at[pl.ds(cnt, LANES)], qv, mask=m)
        plsc.store_compressed(wl1_v.at[pl.ds(cnt, LANES)], pL, mask=m)
        plsc.store_compressed(wr1_v.at[pl.ds(cnt, LANES)], pR, mask=m)
        npop = jnp.max(plsc.all_reduce_population_count(m))
        return cnt + npop

    with jax.named_scope("extract"):
        cnt = lax.fori_loop(0, NG_EXT, _extract, jnp.int32(0))
    nchp = (cnt + PCH - 1) // PCH

    int_min = jnp.int32(_INT_MIN)
    q0 = jnp.max(jnp.where(iota == 0, q1_v[pl.ds(0, LANES)], int_min))
    wl0 = jnp.max(jnp.where(iota == 0, wl1_v[pl.ds(0, LANES)], int_min))
    wr0 = jnp.max(jnp.where(iota == 0, wr1_v[pl.ds(0, LANES)], int_min))

    with jax.named_scope("fill_pad"):
        @pl.loop(0, NG_LIST)
        def _(g):
            keep = (iota + g * LANES) < cnt
            qv = jnp.where(keep, q1_v[pl.ds(g * LANES, LANES)], q0)
            wlv = jnp.where(keep, wl1_v[pl.ds(g * LANES, LANES)], wl0)
            wrv = jnp.where(keep, wr1_v[pl.ds(g * LANES, LANES)], wr0)
            wl1_v[pl.ds(g * LANES, LANES)] = wlv
            wr1_v[pl.ds(g * LANES, LANES)] = wrv
            j = g // (PCH // LANES)
            c = (g % (PCH // LANES)) * LANES
            qp2_v[j, pl.ds(c, LANES)] = qv
            wlp2_v[j, pl.ds(c, LANES)] = jnp.where(
                wlv >= 0, wlv >> 1, qv & (BATP - 1))
            wrp2_v[j, pl.ds(c, LANES)] = jnp.where(
                wrv >= 0, wrv >> 1, qv & (BATP - 1))

    def g_start(j, b):
        pltpu.make_async_copy(val_hbm.at[wlp2_v.at[j]], gl_v.at[b],
                              gsems[b]).start()
        pltpu.make_async_copy(val_hbm.at[wrp2_v.at[j]], gr_v.at[b],
                              gsems[b]).start()
        pltpu.make_async_copy(mem_hbm.at[qp2_v.at[j]], mb_v.at[b],
                              gsems[b]).start()

    def g_wait(j, b):
        pltpu.make_async_copy(val_hbm.at[wlp2_v.at[j]], gl_v.at[b],
                              gsems[b]).wait()
        pltpu.make_async_copy(val_hbm.at[wrp2_v.at[j]], gr_v.at[b],
                              gsems[b]).wait()
        pltpu.make_async_copy(mem_hbm.at[qp2_v.at[j]], mb_v.at[b],
                              gsems[b]).wait()

    with jax.named_scope("gs_prime"):
        @pl.when(nchp > 0)
        def _():
            g_start(0, 0)

        @pl.when(nchp > 1)
        def _():
            g_start(1, 1)

    with jax.named_scope("copy_drain"):
        for c in range(NCC_LAST):
            cin(c, CCHP).wait()
            cout(c, CCHP).start()
            cout(c, CCHP).wait()
            if c + 2 < NCC_LAST:
                cin(c + 2, CCHP).start()
            elif c + 2 == NCC_LAST:
                @pl.when(is_last)
                def _():
                    cin(NCC_LAST, CTAILP_LAST).start()

                @pl.when(jnp.logical_not(is_last))
                def _():
                    cin(NCC_LAST, CCHP).start()
            else:
                @pl.when(jnp.logical_not(is_last))
                def _():
                    cin(NCC_LAST + 1, CCHP).start()

        @pl.when(is_last)
        def _():
            cin(NCC_LAST, CTAILP_LAST).wait()
            cout(NCC_LAST, CTAILP_LAST).start()
            cout(NCC_LAST, CTAILP_LAST).wait()

        @pl.when(jnp.logical_not(is_last))
        def _():
            cin(NCC_LAST, CCHP).wait()
            cout(NCC_LAST, CCHP).start()
            cout(NCC_LAST, CCHP).wait()
            cin(NCC_LAST + 1, CCHP).wait()
            cout(NCC_LAST + 1, CCHP).start()
            cout(NCC_LAST + 1, CCHP).wait()

    def patch(j, b):
        for h, graw, gbuf in ((0, wl1_v, gl_v), (1, wr1_v, gr_v)):
            hbase = jnp.full((LANES,), h * DIM, jnp.int32)
            for grp in range(PCH // LANES):
                rows = iota + grp * LANES
                w = graw[pl.ds(j * PCH + grp * LANES, LANES)]
                mask = w >= 0
                parbase = (w & 1) * DIM

                @pl.loop(0, DIM)
                def _(c):
                    x = plsc.load_gather(gbuf.at[b], [rows, parbase + c])
                    plsc.store_scatter(mb_v.at[b], [rows, hbase + c], x,
                                       mask=mask)

    def handle(j, b):
        @pl.when(j < nchp)
        def _():
            g_wait(j, b)
            patch(j, b)
            pltpu.sync_copy(mb_v.at[b], out_hbm.at[qp2_v.at[j]])

            @pl.when(j + 2 < nchp)
            def _():
                g_start(j + 2, b)

    with jax.named_scope("gs"):
        @pl.loop(0, (MAXCH + 1) // 2)
        def _(k):
            handle(2 * k, 0)
            handle(2 * k + 1, 1)


@jax.jit
def _scatter_sc(mem2, idx32, val2):
    mesh = plsc.VectorSubcoreMesh(core_axis_name="c", subcore_axis_name="s")
    kfn = pl.kernel(
        _sc_body,
        out_type=jax.ShapeDtypeStruct((CAPP, PDIM), mem2.dtype),
        mesh=mesh,
        compiler_params=pltpu.CompilerParams(needs_layout_passes=False),
        scratch_types=[
            pltpu.VMEM((BATCH,), jnp.int32),
            pltpu.VMEM((POS_PAD,), jnp.int32),
            pltpu.VMEM((LIST_PAD,), jnp.int32),
            pltpu.VMEM((LIST_PAD,), jnp.int32),
            pltpu.VMEM((LIST_PAD,), jnp.int32),
            pltpu.VMEM((MAXCH, PCH), jnp.int32),
            pltpu.VMEM((MAXCH, PCH), jnp.int32),
            pltpu.VMEM((MAXCH, PCH), jnp.int32),
            pltpu.VMEM((2, PCH, PDIM), jnp.float32),
            pltpu.VMEM((2, PCH, PDIM), jnp.float32),
            pltpu.VMEM((2, PCH, PDIM), jnp.float32),
            pltpu.VMEM((2, CCHP, PDIM), jnp.float32),
            pltpu.SemaphoreType.DMA,
            pltpu.SemaphoreType.DMA,
            pltpu.SemaphoreType.DMA,
            pltpu.SemaphoreType.DMA,
            pltpu.SemaphoreType.DMA,
            pltpu.SemaphoreType.DMA,
            pltpu.SemaphoreType.DMA,
            pltpu.SemaphoreType.DMA,
        ],
    )
    return kfn(mem2, idx32, val2)


def kernel(mem, idx, val):
    mem2 = mem.reshape(CAPP, PDIM)
    val2 = val.reshape(BATP, PDIM)
    out2 = _scatter_sc(mem2, idx.astype(jnp.int32), val2)
    return out2.reshape(CAP, DIM)


# speedup vs baseline: 1.2816x; 1.2816x over previous
import jax
import jax.numpy as jnp
from jax import lax
from jax.experimental import pallas as pl
from jax.experimental.pallas import tpu as pltpu
from jax.experimental.pallas import tpu_sc as plsc

CAP = 100000
DIM = 64
BATCH = 16384
BATP = BATCH // 2
PDIM = 2 * DIM

NW = 32
SLAB = 3128
SLAB_LAST = CAP - (NW - 1) * SLAB
LANES = 16
POS_PAD = 3136
NG_SLAB = POS_PAD // LANES

BCH = 128
NBK = 23
TAIL_N = SLAB - 24 * BCH
TAIL_L = SLAB_LAST - 23 * BCH

GW = BCH + 8
LIST_SZ = 3264
NG_LIST = LIST_SZ // LANES

_INT_MIN = -2147483647 - 1


def _sc_body(mem_hbm, idx_hbm, val_hbm, out_hbm,
             idx_v, pos_v, row1_v, win1_v, wp1_v, vstage_v, cbuf_v,
             isem0, isem1, osem0, osem1, gsem0, gsem1):
    wid = lax.axis_index("s") * 2 + lax.axis_index("c")
    base = wid * SLAB
    is_last = wid == NW - 1
    slab_len = jnp.where(is_last, SLAB_LAST, SLAB)

    isems = (isem0, isem1)
    osems = (osem0, osem1)
    gsems = (gsem0, gsem1)

    def cin(c, n):
        b = c % 2
        return pltpu.make_async_copy(
            mem_hbm.at[pl.ds(base + c * BCH, n)],
            cbuf_v.at[b, pl.ds(0, n)], isems[b])

    def cout(c, n):
        b = c % 2
        return pltpu.make_async_copy(
            cbuf_v.at[b, pl.ds(0, n)],
            out_hbm.at[pl.ds(base + c * BCH, n)], osems[b])

    cin(0, BCH).start()
    cin(1, BCH).start()

    with jax.named_scope("stage_idx"):
        pltpu.sync_copy(idx_hbm, idx_v)

    iota = lax.iota(jnp.int32, LANES)
    neg1 = jnp.full((LANES,), -1, jnp.int32)

    with jax.named_scope("init_pos"):
        @pl.loop(0, POS_PAD, step=LANES)
        def _(off):
            pos_v[pl.ds(off, LANES)] = neg1

    with jax.named_scope("scan"):
        @pl.loop(0, BATCH, step=LANES)
        def _(off):
            v = idx_v[pl.ds(off, LANES)]
            loc = v - base
            m = (loc >= 0) & (loc < slab_len)
            loc = jnp.where(m, loc, 0)
            plsc.store_scatter(pos_v, [loc], iota + off, mask=m)

    def _extract(g, carry):
        cnt, blo, bhi = carry
        p = pos_v[pl.ds(g * LANES, LANES)]
        m = p >= 0
        rows = iota + g * LANES
        plsc.store_compressed(row1_v.at[pl.ds(cnt, LANES)], rows, mask=m)
        plsc.store_compressed(win1_v.at[pl.ds(cnt, LANES)], p, mask=m)
        npop = plsc.all_reduce_population_count(m)
        bk = g >> 3
        blo = blo + jnp.where(iota == bk, npop, 0)
        bhi = bhi + jnp.where(iota == bk - LANES, npop, 0)
        return cnt + jnp.max(npop), blo, bhi

    zeros = jnp.zeros((LANES,), jnp.int32)
    with jax.named_scope("extract"):
        cnt, blo, bhi = lax.fori_loop(
            0, NG_SLAB, _extract, (jnp.int32(0), zeros, zeros))

    ends_lo = plsc.cumsum(blo)
    tot_lo = jnp.max(jnp.where(iota == LANES - 1, ends_lo, 0))
    ends_hi = plsc.cumsum(bhi) + tot_lo
    starts_lo = ends_lo - blo
    starts_hi = ends_hi - bhi

    int_min = jnp.int32(_INT_MIN)

    def bucket_range(c):
        if c < LANES:
            s = jnp.max(jnp.where(iota == c, starts_lo, int_min))
            e = jnp.max(jnp.where(iota == c, ends_lo, int_min))
        else:
            s = jnp.max(jnp.where(iota == c - LANES, starts_hi, int_min))
            e = jnp.max(jnp.where(iota == c - LANES, ends_hi, int_min))
        return s, e

    with jax.named_scope("fill"):
        @pl.loop(0, NG_LIST)
        def _(g):
            lanepos = iota + g * LANES
            keep = lanepos < cnt
            w = win1_v[pl.ds(g * LANES, LANES)]
            wp1_v[pl.ds(g * LANES, LANES)] = jnp.where(
                keep, lax.shift_right_logical(w, 1), lanepos & (BATP - 1))

    def g_copy(c):
        b = c % 2
        s, _ = bucket_range(c)
        fl8 = pl.multiple_of(s & ~jnp.int32(7), 8)
        return pltpu.make_async_copy(
            val_hbm.at[wp1_v.at[pl.ds(fl8, GW)]], vstage_v.at[b], gsems[b])

    with jax.named_scope("gprime"):
        g_copy(0).start()
        g_copy(1).start()

    def patch(c, n):
        b = c % 2
        s, e = bucket_range(c)
        fl8 = s & ~jnp.int32(7)
        ng = (e - s + LANES - 1) >> 4

        @pl.loop(0, ng)
        def _(t):
            off = s + t * LANES
            lanepos = off + iota
            lmask = lanepos < e
            rowv = row1_v[pl.ds(off, LANES)]
            w = win1_v[pl.ds(off, LANES)]
            parbase = (w & 1) * DIM
            loc = jnp.where(lmask, rowv - c * BCH, 0)
            vrow = jnp.where(lmask, lanepos - fl8, 0)

            @pl.loop(0, DIM)
            def _(col):
                x = plsc.load_gather(vstage_v.at[b], [vrow, parbase + col])
                plsc.store_scatter(cbuf_v.at[b], [loc, zeros + col], x,
                                   mask=lmask)

    with jax.named_scope("drain"):
        for c in range(NBK):
            cin(c, BCH).wait()
            g_copy(c).wait()
            patch(c, BCH)
            if c + 2 <= NBK:
                g_copy(c + 2).start()
            else:
                @pl.when(jnp.logical_not(is_last))
                def _():
                    g_copy(c + 2).start()
            cout(c, BCH).start()
            cout(c, BCH).wait()
            if c + 2 < NBK:
                cin(c + 2, BCH).start()
            elif c + 2 == NBK:
                @pl.when(is_last)
                def _():
                    cin(NBK, TAIL_L).start()

                @pl.when(jnp.logical_not(is_last))
                def _():
                    cin(NBK, BCH).start()
            else:
                @pl.when(jnp.logical_not(is_last))
                def _():
                    cin(NBK + 1, TAIL_N).start()

        @pl.when(is_last)
        def _():
            cin(NBK, TAIL_L).wait()
            g_copy(NBK).wait()
            patch(NBK, TAIL_L)
            cout(NBK, TAIL_L).start()
            cout(NBK, TAIL_L).wait()

        @pl.when(jnp.logical_not(is_last))
        def _():
            cin(NBK, BCH).wait()
            g_copy(NBK).wait()
            patch(NBK, BCH)
            g_copy(NBK + 1).start()
            cout(NBK, BCH).start()
            cout(NBK, BCH).wait()
            cin(NBK + 1, TAIL_N).wait()
            g_copy(NBK + 1).wait()
            patch(NBK + 1, TAIL_N)
            cout(NBK + 1, TAIL_N).start()
            cout(NBK + 1, TAIL_N).wait()


@jax.jit
def _scatter_sc(mem, idx32, val2):
    mesh = plsc.VectorSubcoreMesh(core_axis_name="c", subcore_axis_name="s")
    kfn = pl.kernel(
        _sc_body,
        out_type=jax.ShapeDtypeStruct((CAP, DIM), mem.dtype),
        mesh=mesh,
        compiler_params=pltpu.CompilerParams(needs_layout_passes=False),
        scratch_types=[
            pltpu.VMEM((BATCH,), jnp.int32),
            pltpu.VMEM((POS_PAD,), jnp.int32),
            pltpu.VMEM((LIST_SZ,), jnp.int32),
            pltpu.VMEM((LIST_SZ,), jnp.int32),
            pltpu.VMEM((LIST_SZ,), jnp.int32),
            pltpu.VMEM((2, GW, PDIM), jnp.float32),
            pltpu.VMEM((2, BCH, DIM), jnp.float32),
            pltpu.SemaphoreType.DMA,
            pltpu.SemaphoreType.DMA,
            pltpu.SemaphoreType.DMA,
            pltpu.SemaphoreType.DMA,
            pltpu.SemaphoreType.DMA,
            pltpu.SemaphoreType.DMA,
        ],
    )
    return kfn(mem, idx32, val2)


def kernel(mem, idx, val):
    val2 = val.reshape(BATP, PDIM)
    return _scatter_sc(mem, idx.astype(jnp.int32), val2)
